# baseline (speedup 1.0000x reference)
import functools

import jax
import jax.numpy as jnp
from jax import lax
from jax.experimental import pallas as pl
from jax.experimental.pallas import tpu as pltpu
from jax.experimental.pallas import tpu_sc as plsc

_K = 8192
_D = 256
_B = 1024
_L = 16
_DECAY = 0.9
_THRESH = 1e-4

_BB = 256

_SC_NUM_CORES = 2
_SC_NUM_SUBCORES = 16


def _finish_prev(decay_prev, residual, rows, act, bestp):
    d = jnp.sum(residual * rows, axis=1)
    sign = jnp.where(d >= 0, 1.0, -1.0)
    sidx = jnp.where(d >= 0, bestp, -(bestp + 1))
    w = (act[:, 0] * sign) * decay_prev
    return residual - w[:, None] * rows, sidx, w


def _tc_step_kernel(decay_prev, apply_update, res_ref, row_ref, act_ref,
                    bestp_ref, cb_ref, best_ref, sidx_ref, w_ref, act_out_ref,
                    res_out_ref):
    residual = res_ref[...]
    if apply_update:
        residual, sidx, w = _finish_prev(
            decay_prev, residual, row_ref[...], act_ref[...], bestp_ref[0, :])
        sidx_ref[0, :] = sidx
        w_ref[...] = w[:, None]
    else:
        sidx_ref[0, :] = jnp.zeros((res_ref.shape[0],), jnp.int32)
        w_ref[...] = jnp.zeros((res_ref.shape[0], 1), jnp.float32)
    rn = jnp.sqrt(jnp.sum(residual * residual, axis=1, keepdims=True))
    active = (rn > _THRESH).astype(jnp.float32)
    rnorm = residual / jnp.maximum(rn, 1e-8)
    cos = lax.dot_general(
        rnorm.astype(jnp.bfloat16), cb_ref[...], (((1,), (1,)), ((), ())),
        preferred_element_type=jnp.float32,
        precision=lax.Precision.DEFAULT)
    best = jnp.argmax(jnp.abs(cos), axis=1).astype(jnp.int32)
    best_ref[0, :] = best
    act_out_ref[...] = active
    res_out_ref[...] = residual


def _tc_step(decay_prev, apply_update, residual, rows, act, bestp, cb_bf16):
    row_spec = pl.BlockSpec((_BB, _D), lambda i: (i, 0))
    col_spec = pl.BlockSpec((_BB, 1), lambda i: (i, 0))
    idx_spec = pl.BlockSpec((1, _BB), lambda i: (0, i))
    kern = functools.partial(_tc_step_kernel, decay_prev, apply_update)
    return pl.pallas_call(
        kern,
        grid=(_B // _BB,),
        in_specs=[
            row_spec,
            row_spec,
            col_spec,
            idx_spec,
            pl.BlockSpec((_K, _D), lambda i: (0, 0)),
        ],
        out_specs=[idx_spec, idx_spec, col_spec, col_spec, row_spec],
        out_shape=[
            jax.ShapeDtypeStruct((1, _B), jnp.int32),
            jax.ShapeDtypeStruct((1, _B), jnp.int32),
            jax.ShapeDtypeStruct((_B, 1), jnp.float32),
            jax.ShapeDtypeStruct((_B, 1), jnp.float32),
            jax.ShapeDtypeStruct((_B, _D), jnp.float32),
        ],
    )(residual, rows, act, bestp, cb_bf16)


def _tc_final_kernel(decay_prev, res_ref, act_ref, bestp_ref, *refs):
    row_refs = refs[:_L]
    w_refs = refs[_L:2 * _L - 1]
    sidx_ref, rec_out_ref = refs[2 * _L - 1], refs[2 * _L]
    _, sidx, w_last = _finish_prev(
        decay_prev, res_ref[...], row_refs[-1][...], act_ref[...],
        bestp_ref[0, :])
    sidx_ref[0, :] = sidx
    recon = jnp.zeros_like(res_ref[...])
    for s in range(_L):
        w_s = w_refs[s][...] if s < _L - 1 else w_last[:, None]
        recon = recon + w_s * row_refs[s][...]
    rec_out_ref[...] = recon


def _tc_final(decay_prev, residual, act, bestp, rows_list, w_list):
    row_spec = pl.BlockSpec((_BB, _D), lambda i: (i, 0))
    col_spec = pl.BlockSpec((_BB, 1), lambda i: (i, 0))
    return pl.pallas_call(
        functools.partial(_tc_final_kernel, decay_prev),
        grid=(_B // _BB,),
        in_specs=[row_spec, col_spec, pl.BlockSpec((1, _BB), lambda i: (0, i))]
        + [row_spec] * _L + [col_spec] * (_L - 1),
        out_specs=[
            pl.BlockSpec((1, _BB), lambda i: (0, i)),
            row_spec,
        ],
        out_shape=[
            jax.ShapeDtypeStruct((1, _B), jnp.int32),
            jax.ShapeDtypeStruct((_B, _D), jnp.float32),
        ],
    )(residual, act, bestp, *rows_list, *w_list)


@functools.cache
def _make_sc_gather():
    nw = _SC_NUM_CORES * _SC_NUM_SUBCORES
    b_per_w = _B // nw
    mesh = plsc.VectorSubcoreMesh(core_axis_name="c", subcore_axis_name="s",
                                  num_cores=_SC_NUM_CORES)

    @functools.partial(
        pl.kernel, mesh=mesh,
        out_type=jax.ShapeDtypeStruct((_B, _D), jnp.float32),
        scratch_types=[
            pltpu.VMEM((b_per_w,), jnp.int32),
            pltpu.VMEM((b_per_w, _D), jnp.float32),
            pltpu.SemaphoreType.DMA,
        ],
    )
    def gather(table_hbm, idx_hbm, out_hbm, idx_v, rows_v, sem):
        wid = lax.axis_index("s") * _SC_NUM_CORES + lax.axis_index("c")
        base = wid * b_per_w
        pltpu.sync_copy(idx_hbm.at[pl.ds(base, b_per_w)], idx_v)
        pltpu.async_copy(table_hbm.at[idx_v], rows_v, sem).wait()
        pltpu.sync_copy(rows_v, out_hbm.at[pl.ds(base, b_per_w)])

    return gather


def _sc_gather(table, idx):
    return _make_sc_gather()(table, idx)


@jax.jit
def kernel(targets, codebook):
    cb_bf16 = codebook.astype(jnp.bfloat16)
    residual = targets
    act = jnp.zeros((_B, 1), jnp.float32)
    rows = jnp.zeros((_B, _D), jnp.float32)
    best = jnp.zeros((1, _B), jnp.int32)
    rows_list = []
    w_list = []
    idx_steps = []
    for step in range(_L):
        decay_prev = _DECAY ** step
        best, sidx_prev, w_prev, act, residual = _tc_step(
            decay_prev, step > 0, residual, rows, act, best, cb_bf16)
        if step > 0:
            idx_steps.append(sidx_prev[0])
            w_list.append(w_prev)
        rows = _sc_gather(codebook, best[0])
        rows_list.append(rows)
    sidx_last, recon = _tc_final(
        _DECAY ** _L, residual, act, best, rows_list, w_list)
    idx_steps.append(sidx_last[0])
    signed_indices = jnp.stack(idx_steps, axis=1)
    return signed_indices, recon


# speedup vs baseline: 1.0044x; 1.0044x over previous
import functools

import jax
import jax.numpy as jnp
from jax import lax
from jax.experimental import pallas as pl
from jax.experimental.pallas import tpu as pltpu
from jax.experimental.pallas import tpu_sc as plsc

_K = 8192
_D = 256
_B = 1024
_L = 16
_DECAY = 0.9
_THRESH = 1e-4

_BB = 256

_SC_NUM_CORES = 2
_SC_NUM_SUBCORES = 16


def _finish_prev(decay_prev, residual, rows, act, bestp):
    d = jnp.sum(residual * rows, axis=1)
    sign = jnp.where(d >= 0, 1.0, -1.0)
    sidx = jnp.where(d >= 0, bestp, -(bestp + 1))
    w = (act[:, 0] * sign) * decay_prev
    return residual - w[:, None] * rows, sidx, w


def _tc_step_kernel(decay_prev, apply_update, res_ref, row_ref, act_ref,
                    bestp_ref, cb_ref, best_ref, sidx_ref, w_ref, act_out_ref,
                    res_out_ref):
    cb = cb_ref[...]
    for i in range(_B // _BB):
        sl = pl.ds(i * _BB, _BB)
        residual = res_ref[sl, :]
        if apply_update:
            residual, sidx, w = _finish_prev(
                decay_prev, residual, row_ref[sl, :], act_ref[sl, :],
                bestp_ref[0, sl])
            sidx_ref[0, sl] = sidx
            w_ref[sl, :] = w[:, None]
        else:
            sidx_ref[0, sl] = jnp.zeros((_BB,), jnp.int32)
            w_ref[sl, :] = jnp.zeros((_BB, 1), jnp.float32)
        rn = jnp.sqrt(jnp.sum(residual * residual, axis=1, keepdims=True))
        active = (rn > _THRESH).astype(jnp.float32)
        rnorm = residual / jnp.maximum(rn, 1e-8)
        cos = lax.dot_general(
            rnorm.astype(jnp.bfloat16), cb, (((1,), (1,)), ((), ())),
            preferred_element_type=jnp.float32,
            precision=lax.Precision.DEFAULT)
        best = jnp.argmax(jnp.abs(cos), axis=1).astype(jnp.int32)
        best_ref[0, sl] = best
        act_out_ref[sl, :] = active
        res_out_ref[sl, :] = residual


def _tc_step(decay_prev, apply_update, residual, rows, act, bestp, cb_bf16):
    row_spec = pl.BlockSpec((_B, _D), lambda: (0, 0))
    col_spec = pl.BlockSpec((_B, 1), lambda: (0, 0))
    idx_spec = pl.BlockSpec((1, _B), lambda: (0, 0))
    kern = functools.partial(_tc_step_kernel, decay_prev, apply_update)
    return pl.pallas_call(
        kern,
        in_specs=[
            row_spec,
            row_spec,
            col_spec,
            idx_spec,
            pl.BlockSpec((_K, _D), lambda: (0, 0)),
        ],
        out_specs=[idx_spec, idx_spec, col_spec, col_spec, row_spec],
        out_shape=[
            jax.ShapeDtypeStruct((1, _B), jnp.int32),
            jax.ShapeDtypeStruct((1, _B), jnp.int32),
            jax.ShapeDtypeStruct((_B, 1), jnp.float32),
            jax.ShapeDtypeStruct((_B, 1), jnp.float32),
            jax.ShapeDtypeStruct((_B, _D), jnp.float32),
        ],
    )(residual, rows, act, bestp, cb_bf16)


def _tc_final_kernel(decay_prev, res_ref, act_ref, bestp_ref, *refs):
    row_refs = refs[:_L]
    w_refs = refs[_L:2 * _L - 1]
    sidx_ref, rec_out_ref = refs[2 * _L - 1], refs[2 * _L]
    _, sidx, w_last = _finish_prev(
        decay_prev, res_ref[...], row_refs[-1][...], act_ref[...],
        bestp_ref[0, :])
    sidx_ref[0, :] = sidx
    recon = jnp.zeros_like(res_ref[...])
    for s in range(_L):
        w_s = w_refs[s][...] if s < _L - 1 else w_last[:, None]
        recon = recon + w_s * row_refs[s][...]
    rec_out_ref[...] = recon


def _tc_final(decay_prev, residual, act, bestp, rows_list, w_list):
    row_spec = pl.BlockSpec((_BB, _D), lambda i: (i, 0))
    col_spec = pl.BlockSpec((_BB, 1), lambda i: (i, 0))
    return pl.pallas_call(
        functools.partial(_tc_final_kernel, decay_prev),
        grid=(_B // _BB,),
        in_specs=[row_spec, col_spec, pl.BlockSpec((1, _BB), lambda i: (0, i))]
        + [row_spec] * _L + [col_spec] * (_L - 1),
        out_specs=[
            pl.BlockSpec((1, _BB), lambda i: (0, i)),
            row_spec,
        ],
        out_shape=[
            jax.ShapeDtypeStruct((1, _B), jnp.int32),
            jax.ShapeDtypeStruct((_B, _D), jnp.float32),
        ],
    )(residual, act, bestp, *rows_list, *w_list)


@functools.cache
def _make_sc_gather():
    nw = _SC_NUM_CORES * _SC_NUM_SUBCORES
    b_per_w = _B // nw
    mesh = plsc.VectorSubcoreMesh(core_axis_name="c", subcore_axis_name="s",
                                  num_cores=_SC_NUM_CORES)

    @functools.partial(
        pl.kernel, mesh=mesh,
        out_type=jax.ShapeDtypeStruct((_B, _D), jnp.float32),
        scratch_types=[
            pltpu.VMEM((b_per_w,), jnp.int32),
            pltpu.VMEM((b_per_w, _D), jnp.float32),
            pltpu.SemaphoreType.DMA,
        ],
    )
    def gather(table_hbm, idx_hbm, out_hbm, idx_v, rows_v, sem):
        wid = lax.axis_index("s") * _SC_NUM_CORES + lax.axis_index("c")
        base = wid * b_per_w
        pltpu.sync_copy(idx_hbm.at[pl.ds(base, b_per_w)], idx_v)
        pltpu.async_copy(table_hbm.at[idx_v], rows_v, sem).wait()
        pltpu.sync_copy(rows_v, out_hbm.at[pl.ds(base, b_per_w)])

    return gather


def _sc_gather(table, idx):
    return _make_sc_gather()(table, idx)


@jax.jit
def kernel(targets, codebook):
    cb_bf16 = codebook.astype(jnp.bfloat16)
    residual = targets
    act = jnp.zeros((_B, 1), jnp.float32)
    rows = jnp.zeros((_B, _D), jnp.float32)
    best = jnp.zeros((1, _B), jnp.int32)
    rows_list = []
    w_list = []
    idx_steps = []
    for step in range(_L):
        decay_prev = _DECAY ** step
        best, sidx_prev, w_prev, act, residual = _tc_step(
            decay_prev, step > 0, residual, rows, act, best, cb_bf16)
        if step > 0:
            idx_steps.append(sidx_prev[0])
            w_list.append(w_prev)
        rows = _sc_gather(codebook, best[0])
        rows_list.append(rows)
    sidx_last, recon = _tc_final(
        _DECAY ** _L, residual, act, best, rows_list, w_list)
    idx_steps.append(sidx_last[0])
    signed_indices = jnp.stack(idx_steps, axis=1)
    return signed_indices, recon


# speedup vs baseline: 1.0073x; 1.0029x over previous
import functools

import jax
import jax.numpy as jnp
from jax import lax
from jax.experimental import pallas as pl
from jax.experimental.pallas import tpu as pltpu
from jax.experimental.pallas import tpu_sc as plsc

_K = 8192
_D = 256
_B = 1024
_L = 16
_DECAY = 0.9
_THRESH = 1e-4

_BB = 256

_SC_NUM_CORES = 2
_SC_NUM_SUBCORES = 16


def _finish_prev(decay_prev, res_ref, rec_ref, row_ref, act_ref, best_ref):
    residual = res_ref[...]
    recon = rec_ref[...]
    rows = row_ref[...]
    d = jnp.sum(residual * rows, axis=1)
    sign = jnp.where(d >= 0, 1.0, -1.0)
    bestp = best_ref[0, :]
    sidx = jnp.where(d >= 0, bestp, -(bestp + 1))
    w = (act_ref[..., 0] * sign) * decay_prev
    contribution = w[:, None] * rows
    return residual - contribution, recon + contribution, sidx


def _tc_step_kernel(decay_prev, apply_update, res_ref, rec_ref, row_ref,
                    act_ref, bestp_ref, cb_ref, best_ref, sidx_ref, act_out_ref,
                    res_out_ref, rec_out_ref):
    if apply_update:
        residual, recon, sidx = _finish_prev(
            decay_prev, res_ref, rec_ref, row_ref, act_ref, bestp_ref)
        sidx_ref[0, :] = sidx
    else:
        residual = res_ref[...]
        recon = rec_ref[...]
        sidx_ref[0, :] = jnp.zeros((res_ref.shape[0],), jnp.int32)
    rn = jnp.sqrt(jnp.sum(residual * residual, axis=1, keepdims=True))
    active = (rn > _THRESH).astype(jnp.float32)
    rnorm = residual / jnp.maximum(rn, 1e-8)
    cos = lax.dot_general(
        rnorm.astype(jnp.bfloat16), cb_ref[...], (((1,), (1,)), ((), ())),
        preferred_element_type=jnp.float32,
        precision=lax.Precision.DEFAULT)
    best = jnp.argmax(jnp.abs(cos), axis=1).astype(jnp.int32)
    best_ref[0, :] = best
    act_out_ref[...] = active
    res_out_ref[...] = residual
    rec_out_ref[...] = recon


def _tc_step(decay_prev, apply_update, residual, recon, rows, act, bestp,
             cb_bf16):
    row_spec = pl.BlockSpec((_BB, _D), lambda i: (i, 0))
    col_spec = pl.BlockSpec((_BB, 1), lambda i: (i, 0))
    idx_spec = pl.BlockSpec((1, _BB), lambda i: (0, i))
    kern = functools.partial(_tc_step_kernel, decay_prev, apply_update)
    return pl.pallas_call(
        kern,
        grid=(_B // _BB,),
        in_specs=[
            row_spec,
            row_spec,
            row_spec,
            col_spec,
            idx_spec,
            pl.BlockSpec((_K, _D), lambda i: (0, 0)),
        ],
        out_specs=[idx_spec, idx_spec, col_spec, row_spec, row_spec],
        out_shape=[
            jax.ShapeDtypeStruct((1, _B), jnp.int32),
            jax.ShapeDtypeStruct((1, _B), jnp.int32),
            jax.ShapeDtypeStruct((_B, 1), jnp.float32),
            jax.ShapeDtypeStruct((_B, _D), jnp.float32),
            jax.ShapeDtypeStruct((_B, _D), jnp.float32),
        ],
    )(residual, recon, rows, act, bestp, cb_bf16)


def _tc_final_kernel(decay_prev, res_ref, rec_ref, row_ref, act_ref,
                     bestp_ref, sidx_ref, rec_out_ref):
    _, recon, sidx = _finish_prev(
        decay_prev, res_ref, rec_ref, row_ref, act_ref, bestp_ref)
    sidx_ref[0, :] = sidx
    rec_out_ref[...] = recon


def _tc_final(decay_prev, residual, recon, rows, act, bestp):
    row_spec = pl.BlockSpec((_BB, _D), lambda i: (i, 0))
    return pl.pallas_call(
        functools.partial(_tc_final_kernel, decay_prev),
        grid=(_B // _BB,),
        in_specs=[
            row_spec, row_spec, row_spec,
            pl.BlockSpec((_BB, 1), lambda i: (i, 0)),
            pl.BlockSpec((1, _BB), lambda i: (0, i)),
        ],
        out_specs=[
            pl.BlockSpec((1, _BB), lambda i: (0, i)),
            row_spec,
        ],
        out_shape=[
            jax.ShapeDtypeStruct((1, _B), jnp.int32),
            jax.ShapeDtypeStruct((_B, _D), jnp.float32),
        ],
    )(residual, recon, rows, act, bestp)


@functools.cache
def _make_sc_gather():
    nw = _SC_NUM_CORES * _SC_NUM_SUBCORES
    b_per_w = _B // nw
    mesh = plsc.VectorSubcoreMesh(core_axis_name="c", subcore_axis_name="s",
                                  num_cores=_SC_NUM_CORES)

    @functools.partial(
        pl.kernel, mesh=mesh,
        out_type=jax.ShapeDtypeStruct((_B, _D), jnp.float32),
        scratch_types=[
            pltpu.VMEM((b_per_w,), jnp.int32),
            pltpu.VMEM((b_per_w, _D), jnp.float32),
            pltpu.SemaphoreType.DMA,
        ],
    )
    def gather(table_hbm, idx_hbm, out_hbm, idx_v, rows_v, sem):
        wid = lax.axis_index("s") * _SC_NUM_CORES + lax.axis_index("c")
        base = wid * b_per_w
        pltpu.sync_copy(idx_hbm.at[pl.ds(base, b_per_w)], idx_v)
        pltpu.async_copy(table_hbm.at[idx_v], rows_v, sem).wait()
        pltpu.sync_copy(rows_v, out_hbm.at[pl.ds(base, b_per_w)])

    return gather


def _sc_gather(table, idx):
    return _make_sc_gather()(table, idx)


@jax.jit
def kernel(targets, codebook):
    cb_bf16 = codebook.astype(jnp.bfloat16)
    residual = targets
    recon = jnp.zeros((_B, _D), jnp.float32)
    rows = jnp.zeros((_B, _D), jnp.float32)
    act = jnp.zeros((_B, 1), jnp.float32)
    best = jnp.zeros((1, _B), jnp.int32)
    idx_steps = []
    for step in range(_L):
        decay_prev = _DECAY ** step
        best, sidx_prev, act, residual, recon = _tc_step(
            decay_prev, step > 0, residual, recon, rows, act, best, cb_bf16)
        if step > 0:
            idx_steps.append(sidx_prev[0])
        rows = _sc_gather(codebook, best[0])
    sidx_last, recon = _tc_final(
        _DECAY ** _L, residual, recon, rows, act, best)
    idx_steps.append(sidx_last[0])
    signed_indices = jnp.stack(idx_steps, axis=1)
    return signed_indices, recon
